# Initial kernel scaffold; baseline (speedup 1.0000x reference)
import jax
import jax.numpy as jnp
from jax.experimental import pallas as pl


def kernel(x, edge_index, edge_attr, batch, W1, b1, W2, b2, W3, b3, fcW1, fcb1, fcW2, fcb2):
    raise NotImplementedError("write your pallas kernel here")


# speedup vs baseline: 18.9603x; 18.9603x over previous
import functools

import jax
import jax.numpy as jnp
from jax import lax
from jax.experimental import pallas as pl
from jax.experimental.pallas import tpu as pltpu
from jax.experimental.pallas import tpu_sc as plsc

N = 10000
E = 320000
F = 128
G = 64
PAD = 10240
RPS = PAD // 16
E_ROWS = 2560
NC, NS = 2, 16

_mesh = plsc.VectorSubcoreMesh(
    core_axis_name="c", subcore_axis_name="s", num_cores=NC, num_subcores=NS)


DEG_RPW = E_ROWS // (NC * NS)


@functools.partial(
    pl.kernel,
    out_type=jax.ShapeDtypeStruct((NC, PAD), jnp.float32),
    mesh=_mesh,
    scratch_types=[
        pltpu.VMEM_SHARED((PAD,), jnp.float32),
        pltpu.VMEM((DEG_RPW, 128), jnp.int32),
        pltpu.VMEM((RPS,), jnp.float32),
        pltpu.VMEM((128,), jnp.float32),
    ],
)
def _sc_degree(dst_h, deg_h, acc_sh, dst_t, zbuf, ones_t):
    c = lax.axis_index("c")
    s = lax.axis_index("s")
    w = c * NS + s

    def fill(i, _):
        zbuf[pl.ds(i * 16, 16)] = jnp.zeros((16,), jnp.float32)
        ones_t[pl.ds((i % 8) * 16, 16)] = jnp.ones((16,), jnp.float32)
        return 0
    lax.fori_loop(0, RPS // 16, fill, 0)
    pltpu.sync_copy(zbuf, acc_sh.at[pl.ds(s * RPS, RPS)])
    pltpu.sync_copy(dst_h.at[pl.ds(w * DEG_RPW, DEG_RPW)], dst_t)

    plsc.subcore_barrier()

    def body(j, _):
        pltpu.sync_copy(ones_t, acc_sh.at[dst_t.at[j]], add=True)
        return 0
    lax.fori_loop(0, DEG_RPW, body, 0)

    plsc.subcore_barrier()
    pltpu.sync_copy(acc_sh.at[pl.ds(s * RPS, RPS)], deg_h.at[c, pl.ds(s * RPS, RPS)])


AGG_RPW = E_ROWS // (NC * NS)
IDXC = 8


@functools.partial(
    pl.kernel,
    out_type=jax.ShapeDtypeStruct((NC, PAD, F), jnp.float32),
    mesh=_mesh,
    scratch_types=[
        pltpu.VMEM_SHARED((PAD, F), jnp.float32),
        pltpu.VMEM((IDXC, 128), jnp.int32),
        pltpu.VMEM((IDXC, 128), jnp.int32),
        pltpu.VMEM((128, F), jnp.float32),
        pltpu.SemaphoreType.DMA,
    ],
)
def _sc_aggregate(g_h, src_h, dst_h, out_h,
                  acc_sh, src_t, dst_t, rows, sem):
    c = lax.axis_index("c")
    s = lax.axis_index("s")

    pltpu.sync_copy(g_h.at[pl.ds(s * RPS, RPS)], acc_sh.at[pl.ds(s * RPS, RPS)])

    plsc.subcore_barrier()

    base = (c * NS + s) * AGG_RPW

    def outer(blk, _):
        off = pl.multiple_of(base + blk * IDXC, 8)
        pltpu.sync_copy(src_h.at[pl.ds(off, IDXC)], src_t)
        pltpu.sync_copy(dst_h.at[pl.ds(off, IDXC)], dst_t)

        def body(j, _):
            pltpu.async_copy(g_h.at[src_t.at[j]], rows, sem).wait()
            pltpu.sync_copy(rows, acc_sh.at[dst_t.at[j]], add=True)
            return 0
        lax.fori_loop(0, IDXC, body, 0)
        return 0
    lax.fori_loop(0, AGG_RPW // IDXC, outer, 0)

    plsc.subcore_barrier()
    pltpu.sync_copy(acc_sh.at[pl.ds(s * RPS, RPS)],
                    out_h.at[c, pl.ds(s * RPS, RPS)])


BLK = PAD // 8


def _prep_body(deg_ref, x_ref, w_ref, g_ref, dinv_ref):
    tot = deg_ref[0] + deg_ref[1] + 1.0
    di = lax.rsqrt(tot)
    h = jnp.dot(x_ref[...], w_ref[...], preferred_element_type=jnp.float32)
    g_ref[...] = h * di[:, None]
    dinv_ref[0, 0] = di


def _tc_prep(deg, x_pad, W1):
    return pl.pallas_call(
        _prep_body,
        grid=(8,),
        in_specs=[
            pl.BlockSpec((NC, BLK), lambda i: (0, i)),
            pl.BlockSpec((BLK, F), lambda i: (i, 0)),
            pl.BlockSpec((F, F), lambda i: (0, 0)),
        ],
        out_specs=[
            pl.BlockSpec((BLK, F), lambda i: (i, 0)),
            pl.BlockSpec((1, 1, BLK), lambda i: (i, 0, 0)),
        ],
        out_shape=[
            jax.ShapeDtypeStruct((PAD, F), jnp.float32),
            jax.ShapeDtypeStruct((8, 1, BLK), jnp.float32),
        ],
    )(deg, x_pad, W1)


def _mid_body(s_ref, g_ref, dinv_ref, b_ref, w_ref, gn_ref):
    tot = s_ref[0] + s_ref[1] - g_ref[...]
    di = dinv_ref[0, 0]
    act = jnp.maximum(tot * di[:, None] + b_ref[0][None, :], 0.0)
    h = jnp.dot(act, w_ref[...], preferred_element_type=jnp.float32)
    gn_ref[...] = h * di[:, None]


def _tc_mid(s_agg, g, dinv, b_prev, W_next):
    return pl.pallas_call(
        _mid_body,
        grid=(8,),
        in_specs=[
            pl.BlockSpec((NC, BLK, F), lambda i: (0, i, 0)),
            pl.BlockSpec((BLK, F), lambda i: (i, 0)),
            pl.BlockSpec((1, 1, BLK), lambda i: (i, 0, 0)),
            pl.BlockSpec((1, F), lambda i: (0, 0)),
            pl.BlockSpec((F, F), lambda i: (0, 0)),
        ],
        out_specs=pl.BlockSpec((BLK, F), lambda i: (i, 0)),
        out_shape=jax.ShapeDtypeStruct((PAD, F), jnp.float32),
    )(s_agg, g, dinv, b_prev.reshape(1, F), W_next)


def _tail_body(s_ref, g_ref, dinv_ref, b_ref, batch_ref, fw1_ref, fb1_ref,
               fw2_ref, fb2_ref, out_ref, sums_sc, cnts_sc):
    i = pl.program_id(0)
    tot = s_ref[0] + s_ref[1] - g_ref[...]
    di = dinv_ref[0, 0]
    act = jnp.maximum(tot * di[:, None] + b_ref[0][None, :], 0.0)
    bids = batch_ref[0, 0]
    onehot = (bids[:, None] ==
              lax.broadcasted_iota(jnp.int32, (BLK, G), 1)).astype(jnp.float32)
    part = lax.dot_general(onehot, act, (((0,), (0,)), ((), ())),
                           preferred_element_type=jnp.float32)
    cnt = jnp.sum(onehot, axis=0)

    @pl.when(i == 0)
    def _():
        sums_sc[...] = jnp.zeros_like(sums_sc)
        cnts_sc[...] = jnp.zeros_like(cnts_sc)
    sums_sc[...] += part
    cnts_sc[0] += cnt

    @pl.when(i == 7)
    def _():
        pooled = sums_sc[...] / jnp.maximum(cnts_sc[0], 1.0)[:, None]
        o1 = jnp.maximum(
            jnp.dot(pooled, fw1_ref[...], preferred_element_type=jnp.float32)
            + fb1_ref[0][None, :], 0.0)
        o2 = (jnp.dot(o1, fw2_ref[...], preferred_element_type=jnp.float32)
              + fb2_ref[0][None, :])
        out_ref[...] = o2


def _tc_tail(s_agg, g, dinv, b3, batch_pad, fcW1, fcb1, fcW2, fcb2):
    return pl.pallas_call(
        _tail_body,
        grid=(8,),
        in_specs=[
            pl.BlockSpec((NC, BLK, F), lambda i: (0, i, 0)),
            pl.BlockSpec((BLK, F), lambda i: (i, 0)),
            pl.BlockSpec((1, 1, BLK), lambda i: (i, 0, 0)),
            pl.BlockSpec((1, F), lambda i: (0, 0)),
            pl.BlockSpec((1, 1, BLK), lambda i: (i, 0, 0)),
            pl.BlockSpec((F, 32), lambda i: (0, 0)),
            pl.BlockSpec((1, 32), lambda i: (0, 0)),
            pl.BlockSpec((32, 1), lambda i: (0, 0)),
            pl.BlockSpec((1, 1), lambda i: (0, 0)),
        ],
        out_specs=pl.BlockSpec((G, 1), lambda i: (0, 0)),
        out_shape=jax.ShapeDtypeStruct((G, 1), jnp.float32),
        scratch_shapes=[
            pltpu.VMEM((G, F), jnp.float32),
            pltpu.VMEM((1, G), jnp.float32),
        ],
    )(s_agg, g, dinv, b3.reshape(1, F), batch_pad, fcW1,
      fcb1.reshape(1, 32), fcW2, fcb2.reshape(1, 1))


def kernel(x, edge_index, edge_attr, batch,
           W1, b1, W2, b2, W3, b3, fcW1, fcb1, fcW2, fcb2):
    n_fill = E_ROWS * 128 - E
    fill = (N + jnp.arange(n_fill, dtype=jnp.int32) % (PAD - N))
    src = jnp.concatenate([edge_index[0], fill]).reshape(E_ROWS, 128)
    dst = jnp.concatenate([edge_index[1], fill]).reshape(E_ROWS, 128)

    deg = _sc_degree(dst)
    x_pad = jnp.zeros((PAD, F), jnp.float32).at[:N].set(x)
    g1, dinv = _tc_prep(deg, x_pad, W1)

    s1 = _sc_aggregate(g1, src, dst)
    g2 = _tc_mid(s1, g1, dinv, b1, W2)
    s2 = _sc_aggregate(g2, src, dst)
    g3 = _tc_mid(s2, g2, dinv, b2, W3)
    s3 = _sc_aggregate(g3, src, dst)

    batch_pad = jnp.full((PAD,), G, jnp.int32).at[:N].set(batch).reshape(8, 1, BLK)
    return _tc_tail(s3, g3, dinv, b3, batch_pad, fcW1, fcb1, fcW2, fcb2)


# speedup vs baseline: 22.1410x; 1.1678x over previous
import functools

import jax
import jax.numpy as jnp
from jax import lax
from jax.experimental import pallas as pl
from jax.experimental.pallas import tpu as pltpu
from jax.experimental.pallas import tpu_sc as plsc

N = 10000
E = 320000
F = 128
G = 64
PAD = 10240
RPS = PAD // 16
E_ROWS = 2560
NC, NS = 2, 16

_mesh = plsc.VectorSubcoreMesh(
    core_axis_name="c", subcore_axis_name="s", num_cores=NC, num_subcores=NS)


DEG_RPW = E_ROWS // (NC * NS)


@functools.partial(
    pl.kernel,
    out_type=jax.ShapeDtypeStruct((NC, PAD), jnp.float32),
    mesh=_mesh,
    scratch_types=[
        pltpu.VMEM_SHARED((PAD,), jnp.float32),
        pltpu.VMEM((DEG_RPW, 128), jnp.int32),
        pltpu.VMEM((RPS,), jnp.float32),
        pltpu.VMEM((128,), jnp.float32),
    ],
)
def _sc_degree(dst_h, deg_h, acc_sh, dst_t, zbuf, ones_t):
    c = lax.axis_index("c")
    s = lax.axis_index("s")
    w = c * NS + s

    def fill(i, _):
        zbuf[pl.ds(i * 16, 16)] = jnp.zeros((16,), jnp.float32)
        ones_t[pl.ds((i % 8) * 16, 16)] = jnp.ones((16,), jnp.float32)
        return 0
    lax.fori_loop(0, RPS // 16, fill, 0)
    pltpu.sync_copy(zbuf, acc_sh.at[pl.ds(s * RPS, RPS)])
    pltpu.sync_copy(dst_h.at[pl.ds(w * DEG_RPW, DEG_RPW)], dst_t)

    plsc.subcore_barrier()

    def body(j, _):
        pltpu.sync_copy(ones_t, acc_sh.at[dst_t.at[j]], add=True)
        return 0
    lax.fori_loop(0, DEG_RPW, body, 0)

    plsc.subcore_barrier()
    pltpu.sync_copy(acc_sh.at[pl.ds(s * RPS, RPS)], deg_h.at[c, pl.ds(s * RPS, RPS)])


AGG_RPW = E_ROWS // (NC * NS)
IDXC = 16


@functools.partial(
    pl.kernel,
    out_type=jax.ShapeDtypeStruct((NC, PAD, F), jnp.float32),
    mesh=_mesh,
    scratch_types=[
        pltpu.VMEM_SHARED((PAD, F), jnp.float32),
        pltpu.VMEM((IDXC, 128), jnp.int32),
        pltpu.VMEM((IDXC, 128), jnp.int32),
        pltpu.VMEM((128, F), jnp.float32),
        pltpu.VMEM((128, F), jnp.float32),
        pltpu.SemaphoreType.DMA,
        pltpu.SemaphoreType.DMA,
        pltpu.SemaphoreType.DMA,
        pltpu.SemaphoreType.DMA,
    ],
)
def _sc_aggregate(g_h, src_h, dst_h, out_h,
                  acc_sh, src_t, dst_t, rows_a, rows_b,
                  gsem_a, gsem_b, ssem_a, ssem_b):
    c = lax.axis_index("c")
    s = lax.axis_index("s")

    pltpu.sync_copy(g_h.at[pl.ds(s * RPS, RPS)], acc_sh.at[pl.ds(s * RPS, RPS)])

    plsc.subcore_barrier()

    base = (c * NS + s) * AGG_RPW

    def drain(rows, ssem):
        pltpu.make_async_copy(rows, acc_sh.at[dst_t.at[0]], ssem).wait()

    def outer(blk, _):
        @pl.when(blk > 0)
        def _():
            drain(rows_a, ssem_a)
            drain(rows_b, ssem_b)
        off = pl.multiple_of(base + blk * IDXC, 8)
        pltpu.sync_copy(src_h.at[pl.ds(off, IDXC)], src_t)
        pltpu.sync_copy(dst_h.at[pl.ds(off, IDXC)], dst_t)

        def pair(pj, _):
            j0 = pj * 2
            j1 = j0 + 1

            @pl.when(pj > 0)
            def _():
                drain(rows_a, ssem_a)
                drain(rows_b, ssem_b)
            ga = pltpu.async_copy(g_h.at[src_t.at[j0]], rows_a, gsem_a)
            gb = pltpu.async_copy(g_h.at[src_t.at[j1]], rows_b, gsem_b)
            ga.wait()
            pltpu.async_copy(rows_a, acc_sh.at[dst_t.at[j0]], ssem_a, add=True)
            gb.wait()
            pltpu.async_copy(rows_b, acc_sh.at[dst_t.at[j1]], ssem_b, add=True)
            return 0
        lax.fori_loop(0, IDXC // 2, pair, 0)
        return 0
    lax.fori_loop(0, AGG_RPW // IDXC, outer, 0)

    drain(rows_a, ssem_a)
    drain(rows_b, ssem_b)

    plsc.subcore_barrier()
    pltpu.sync_copy(acc_sh.at[pl.ds(s * RPS, RPS)],
                    out_h.at[c, pl.ds(s * RPS, RPS)])


BLK = PAD // 8


def _prep_body(deg_ref, x_ref, w_ref, g_ref, dinv_ref):
    tot = deg_ref[0] + deg_ref[1] + 1.0
    di = lax.rsqrt(tot)
    h = jnp.dot(x_ref[...], w_ref[...], preferred_element_type=jnp.float32)
    g_ref[...] = h * di[:, None]
    dinv_ref[0, 0] = di


def _tc_prep(deg, x_pad, W1):
    return pl.pallas_call(
        _prep_body,
        grid=(8,),
        in_specs=[
            pl.BlockSpec((NC, BLK), lambda i: (0, i)),
            pl.BlockSpec((BLK, F), lambda i: (i, 0)),
            pl.BlockSpec((F, F), lambda i: (0, 0)),
        ],
        out_specs=[
            pl.BlockSpec((BLK, F), lambda i: (i, 0)),
            pl.BlockSpec((1, 1, BLK), lambda i: (i, 0, 0)),
        ],
        out_shape=[
            jax.ShapeDtypeStruct((PAD, F), jnp.float32),
            jax.ShapeDtypeStruct((8, 1, BLK), jnp.float32),
        ],
    )(deg, x_pad, W1)


def _mid_body(s_ref, g_ref, dinv_ref, b_ref, w_ref, gn_ref):
    tot = s_ref[0] + s_ref[1] - g_ref[...]
    di = dinv_ref[0, 0]
    act = jnp.maximum(tot * di[:, None] + b_ref[0][None, :], 0.0)
    h = jnp.dot(act, w_ref[...], preferred_element_type=jnp.float32)
    gn_ref[...] = h * di[:, None]


def _tc_mid(s_agg, g, dinv, b_prev, W_next):
    return pl.pallas_call(
        _mid_body,
        grid=(8,),
        in_specs=[
            pl.BlockSpec((NC, BLK, F), lambda i: (0, i, 0)),
            pl.BlockSpec((BLK, F), lambda i: (i, 0)),
            pl.BlockSpec((1, 1, BLK), lambda i: (i, 0, 0)),
            pl.BlockSpec((1, F), lambda i: (0, 0)),
            pl.BlockSpec((F, F), lambda i: (0, 0)),
        ],
        out_specs=pl.BlockSpec((BLK, F), lambda i: (i, 0)),
        out_shape=jax.ShapeDtypeStruct((PAD, F), jnp.float32),
    )(s_agg, g, dinv, b_prev.reshape(1, F), W_next)


def _tail_body(s_ref, g_ref, dinv_ref, b_ref, batch_ref, fw1_ref, fb1_ref,
               fw2_ref, fb2_ref, out_ref, sums_sc, cnts_sc):
    i = pl.program_id(0)
    tot = s_ref[0] + s_ref[1] - g_ref[...]
    di = dinv_ref[0, 0]
    act = jnp.maximum(tot * di[:, None] + b_ref[0][None, :], 0.0)
    bids = batch_ref[0, 0]
    onehot = (bids[:, None] ==
              lax.broadcasted_iota(jnp.int32, (BLK, G), 1)).astype(jnp.float32)
    part = lax.dot_general(onehot, act, (((0,), (0,)), ((), ())),
                           preferred_element_type=jnp.float32)
    cnt = jnp.sum(onehot, axis=0)

    @pl.when(i == 0)
    def _():
        sums_sc[...] = jnp.zeros_like(sums_sc)
        cnts_sc[...] = jnp.zeros_like(cnts_sc)
    sums_sc[...] += part
    cnts_sc[0] += cnt

    @pl.when(i == 7)
    def _():
        pooled = sums_sc[...] / jnp.maximum(cnts_sc[0], 1.0)[:, None]
        o1 = jnp.maximum(
            jnp.dot(pooled, fw1_ref[...], preferred_element_type=jnp.float32)
            + fb1_ref[0][None, :], 0.0)
        o2 = (jnp.dot(o1, fw2_ref[...], preferred_element_type=jnp.float32)
              + fb2_ref[0][None, :])
        out_ref[...] = o2


def _tc_tail(s_agg, g, dinv, b3, batch_pad, fcW1, fcb1, fcW2, fcb2):
    return pl.pallas_call(
        _tail_body,
        grid=(8,),
        in_specs=[
            pl.BlockSpec((NC, BLK, F), lambda i: (0, i, 0)),
            pl.BlockSpec((BLK, F), lambda i: (i, 0)),
            pl.BlockSpec((1, 1, BLK), lambda i: (i, 0, 0)),
            pl.BlockSpec((1, F), lambda i: (0, 0)),
            pl.BlockSpec((1, 1, BLK), lambda i: (i, 0, 0)),
            pl.BlockSpec((F, 32), lambda i: (0, 0)),
            pl.BlockSpec((1, 32), lambda i: (0, 0)),
            pl.BlockSpec((32, 1), lambda i: (0, 0)),
            pl.BlockSpec((1, 1), lambda i: (0, 0)),
        ],
        out_specs=pl.BlockSpec((G, 1), lambda i: (0, 0)),
        out_shape=jax.ShapeDtypeStruct((G, 1), jnp.float32),
        scratch_shapes=[
            pltpu.VMEM((G, F), jnp.float32),
            pltpu.VMEM((1, G), jnp.float32),
        ],
    )(s_agg, g, dinv, b3.reshape(1, F), batch_pad, fcW1,
      fcb1.reshape(1, 32), fcW2, fcb2.reshape(1, 1))


def kernel(x, edge_index, edge_attr, batch,
           W1, b1, W2, b2, W3, b3, fcW1, fcb1, fcW2, fcb2):
    n_fill = E_ROWS * 128 - E
    fill = (N + jnp.arange(n_fill, dtype=jnp.int32) % (PAD - N))
    src = jnp.concatenate([edge_index[0], fill]).reshape(E_ROWS, 128)
    dst = jnp.concatenate([edge_index[1], fill]).reshape(E_ROWS, 128)

    deg = _sc_degree(dst)
    x_pad = jnp.zeros((PAD, F), jnp.float32).at[:N].set(x)
    g1, dinv = _tc_prep(deg, x_pad, W1)

    s1 = _sc_aggregate(g1, src, dst)
    g2 = _tc_mid(s1, g1, dinv, b1, W2)
    s2 = _sc_aggregate(g2, src, dst)
    g3 = _tc_mid(s2, g2, dinv, b2, W3)
    s3 = _sc_aggregate(g3, src, dst)

    batch_pad = jnp.full((PAD,), G, jnp.int32).at[:N].set(batch).reshape(8, 1, BLK)
    return _tc_tail(s3, g3, dinv, b3, batch_pad, fcW1, fcb1, fcW2, fcb2)


# speedup vs baseline: 22.3011x; 1.0072x over previous
import functools

import jax
import jax.numpy as jnp
from jax import lax
from jax.experimental import pallas as pl
from jax.experimental.pallas import tpu as pltpu
from jax.experimental.pallas import tpu_sc as plsc

N = 10000
E = 320000
F = 128
G = 64
PAD = 10240
RPS = PAD // 16
E_ROWS = 2560
NC, NS = 2, 16

_mesh = plsc.VectorSubcoreMesh(
    core_axis_name="c", subcore_axis_name="s", num_cores=NC, num_subcores=NS)


DEG_RPW = E_ROWS // (NC * NS)


@functools.partial(
    pl.kernel,
    out_type=jax.ShapeDtypeStruct((NC, PAD), jnp.float32),
    mesh=_mesh,
    scratch_types=[
        pltpu.VMEM_SHARED((PAD,), jnp.float32),
        pltpu.VMEM((DEG_RPW, 128), jnp.int32),
        pltpu.VMEM((RPS,), jnp.float32),
        pltpu.VMEM((128,), jnp.float32),
    ],
)
def _sc_degree(dst_h, deg_h, acc_sh, dst_t, zbuf, ones_t):
    c = lax.axis_index("c")
    s = lax.axis_index("s")
    w = c * NS + s

    def fill(i, _):
        zbuf[pl.ds(i * 16, 16)] = jnp.zeros((16,), jnp.float32)
        ones_t[pl.ds((i % 8) * 16, 16)] = jnp.ones((16,), jnp.float32)
        return 0
    lax.fori_loop(0, RPS // 16, fill, 0)
    pltpu.sync_copy(zbuf, acc_sh.at[pl.ds(s * RPS, RPS)])
    pltpu.sync_copy(dst_h.at[pl.ds(w * DEG_RPW, DEG_RPW)], dst_t)

    plsc.subcore_barrier()

    def body(j, _):
        pltpu.sync_copy(ones_t, acc_sh.at[dst_t.at[j]], add=True)
        return 0
    lax.fori_loop(0, DEG_RPW, body, 0)

    plsc.subcore_barrier()
    pltpu.sync_copy(acc_sh.at[pl.ds(s * RPS, RPS)], deg_h.at[c, pl.ds(s * RPS, RPS)])


E_ROWS64 = E_ROWS * 2
AGG_RPW = E_ROWS64 // (NC * NS)
IDXC = 32
NBUF = 4


@functools.partial(
    pl.kernel,
    out_type=jax.ShapeDtypeStruct((NC, PAD, F), jnp.float32),
    mesh=_mesh,
    scratch_types=[
        pltpu.VMEM_SHARED((PAD, F), jnp.float32),
        pltpu.VMEM((IDXC, 64), jnp.int32),
        pltpu.VMEM((IDXC, 64), jnp.int32),
        [pltpu.VMEM((64, F), jnp.float32)] * NBUF,
        [pltpu.SemaphoreType.DMA] * NBUF,
        [pltpu.SemaphoreType.DMA] * NBUF,
    ],
)
def _sc_aggregate(g_h, src_h, dst_h, out_h,
                  acc_sh, src_t, dst_t, rows, gsem, ssem):
    c = lax.axis_index("c")
    s = lax.axis_index("s")

    pltpu.sync_copy(g_h.at[pl.ds(s * RPS, RPS)], acc_sh.at[pl.ds(s * RPS, RPS)])

    plsc.subcore_barrier()

    base = (c * NS + s) * AGG_RPW

    def drain(k):
        pltpu.make_async_copy(rows[k], acc_sh.at[dst_t.at[0]], ssem[k]).wait()

    def outer(blk, _):
        @pl.when(blk > 0)
        def _():
            for k in range(NBUF):
                drain(k)
        off = pl.multiple_of(base + blk * IDXC, 8)
        pltpu.sync_copy(src_h.at[pl.ds(off, IDXC)], src_t)
        pltpu.sync_copy(dst_h.at[pl.ds(off, IDXC)], dst_t)

        def quad(qj, _):
            j0 = qj * NBUF

            @pl.when(qj > 0)
            def _():
                for k in range(NBUF):
                    drain(k)
            gds = [pltpu.async_copy(g_h.at[src_t.at[j0 + k]], rows[k], gsem[k])
                   for k in range(NBUF)]
            for k in range(NBUF):
                gds[k].wait()
                pltpu.async_copy(rows[k], acc_sh.at[dst_t.at[j0 + k]],
                                 ssem[k], add=True)
            return 0
        lax.fori_loop(0, IDXC // NBUF, quad, 0)
        return 0
    lax.fori_loop(0, AGG_RPW // IDXC, outer, 0)

    for k in range(NBUF):
        drain(k)

    plsc.subcore_barrier()
    pltpu.sync_copy(acc_sh.at[pl.ds(s * RPS, RPS)],
                    out_h.at[c, pl.ds(s * RPS, RPS)])


BLK = PAD // 8


def _prep_body(deg_ref, x_ref, w_ref, g_ref, dinv_ref):
    tot = deg_ref[0] + deg_ref[1] + 1.0
    di = lax.rsqrt(tot)
    h = jnp.dot(x_ref[...], w_ref[...], preferred_element_type=jnp.float32)
    g_ref[...] = h * di[:, None]
    dinv_ref[0, 0] = di


def _tc_prep(deg, x_pad, W1):
    return pl.pallas_call(
        _prep_body,
        grid=(8,),
        in_specs=[
            pl.BlockSpec((NC, BLK), lambda i: (0, i)),
            pl.BlockSpec((BLK, F), lambda i: (i, 0)),
            pl.BlockSpec((F, F), lambda i: (0, 0)),
        ],
        out_specs=[
            pl.BlockSpec((BLK, F), lambda i: (i, 0)),
            pl.BlockSpec((1, 1, BLK), lambda i: (i, 0, 0)),
        ],
        out_shape=[
            jax.ShapeDtypeStruct((PAD, F), jnp.float32),
            jax.ShapeDtypeStruct((8, 1, BLK), jnp.float32),
        ],
    )(deg, x_pad, W1)


def _mid_body(s_ref, g_ref, dinv_ref, b_ref, w_ref, gn_ref):
    tot = s_ref[0] + s_ref[1] - g_ref[...]
    di = dinv_ref[0, 0]
    act = jnp.maximum(tot * di[:, None] + b_ref[0][None, :], 0.0)
    h = jnp.dot(act, w_ref[...], preferred_element_type=jnp.float32)
    gn_ref[...] = h * di[:, None]


def _tc_mid(s_agg, g, dinv, b_prev, W_next):
    return pl.pallas_call(
        _mid_body,
        grid=(8,),
        in_specs=[
            pl.BlockSpec((NC, BLK, F), lambda i: (0, i, 0)),
            pl.BlockSpec((BLK, F), lambda i: (i, 0)),
            pl.BlockSpec((1, 1, BLK), lambda i: (i, 0, 0)),
            pl.BlockSpec((1, F), lambda i: (0, 0)),
            pl.BlockSpec((F, F), lambda i: (0, 0)),
        ],
        out_specs=pl.BlockSpec((BLK, F), lambda i: (i, 0)),
        out_shape=jax.ShapeDtypeStruct((PAD, F), jnp.float32),
    )(s_agg, g, dinv, b_prev.reshape(1, F), W_next)


def _tail_body(s_ref, g_ref, dinv_ref, b_ref, batch_ref, fw1_ref, fb1_ref,
               fw2_ref, fb2_ref, out_ref, sums_sc, cnts_sc):
    i = pl.program_id(0)
    tot = s_ref[0] + s_ref[1] - g_ref[...]
    di = dinv_ref[0, 0]
    act = jnp.maximum(tot * di[:, None] + b_ref[0][None, :], 0.0)
    bids = batch_ref[0, 0]
    onehot = (bids[:, None] ==
              lax.broadcasted_iota(jnp.int32, (BLK, G), 1)).astype(jnp.float32)
    part = lax.dot_general(onehot, act, (((0,), (0,)), ((), ())),
                           preferred_element_type=jnp.float32)
    cnt = jnp.sum(onehot, axis=0)

    @pl.when(i == 0)
    def _():
        sums_sc[...] = jnp.zeros_like(sums_sc)
        cnts_sc[...] = jnp.zeros_like(cnts_sc)
    sums_sc[...] += part
    cnts_sc[0] += cnt

    @pl.when(i == 7)
    def _():
        pooled = sums_sc[...] / jnp.maximum(cnts_sc[0], 1.0)[:, None]
        o1 = jnp.maximum(
            jnp.dot(pooled, fw1_ref[...], preferred_element_type=jnp.float32)
            + fb1_ref[0][None, :], 0.0)
        o2 = (jnp.dot(o1, fw2_ref[...], preferred_element_type=jnp.float32)
              + fb2_ref[0][None, :])
        out_ref[...] = o2


def _tc_tail(s_agg, g, dinv, b3, batch_pad, fcW1, fcb1, fcW2, fcb2):
    return pl.pallas_call(
        _tail_body,
        grid=(8,),
        in_specs=[
            pl.BlockSpec((NC, BLK, F), lambda i: (0, i, 0)),
            pl.BlockSpec((BLK, F), lambda i: (i, 0)),
            pl.BlockSpec((1, 1, BLK), lambda i: (i, 0, 0)),
            pl.BlockSpec((1, F), lambda i: (0, 0)),
            pl.BlockSpec((1, 1, BLK), lambda i: (i, 0, 0)),
            pl.BlockSpec((F, 32), lambda i: (0, 0)),
            pl.BlockSpec((1, 32), lambda i: (0, 0)),
            pl.BlockSpec((32, 1), lambda i: (0, 0)),
            pl.BlockSpec((1, 1), lambda i: (0, 0)),
        ],
        out_specs=pl.BlockSpec((G, 1), lambda i: (0, 0)),
        out_shape=jax.ShapeDtypeStruct((G, 1), jnp.float32),
        scratch_shapes=[
            pltpu.VMEM((G, F), jnp.float32),
            pltpu.VMEM((1, G), jnp.float32),
        ],
    )(s_agg, g, dinv, b3.reshape(1, F), batch_pad, fcW1,
      fcb1.reshape(1, 32), fcW2, fcb2.reshape(1, 1))


def kernel(x, edge_index, edge_attr, batch,
           W1, b1, W2, b2, W3, b3, fcW1, fcb1, fcW2, fcb2):
    n_fill = E_ROWS * 128 - E
    fill = (N + jnp.arange(n_fill, dtype=jnp.int32) % (PAD - N))
    src = jnp.concatenate([edge_index[0], fill]).reshape(E_ROWS64, 64)
    dst = jnp.concatenate([edge_index[1], fill]).reshape(E_ROWS64, 64)

    deg = _sc_degree(dst.reshape(E_ROWS, 128))
    x_pad = jnp.zeros((PAD, F), jnp.float32).at[:N].set(x)
    g1, dinv = _tc_prep(deg, x_pad, W1)

    s1 = _sc_aggregate(g1, src, dst)
    g2 = _tc_mid(s1, g1, dinv, b1, W2)
    s2 = _sc_aggregate(g2, src, dst)
    g3 = _tc_mid(s2, g2, dinv, b2, W3)
    s3 = _sc_aggregate(g3, src, dst)

    batch_pad = jnp.full((PAD,), G, jnp.int32).at[:N].set(batch).reshape(8, 1, BLK)
    return _tc_tail(s3, g3, dinv, b3, batch_pad, fcW1, fcb1, fcW2, fcb2)


# speedup vs baseline: 27.0706x; 1.2139x over previous
import functools

import jax
import jax.numpy as jnp
from jax import lax
from jax.experimental import pallas as pl
from jax.experimental.pallas import tpu as pltpu
from jax.experimental.pallas import tpu_sc as plsc

N = 10000
E = 320000
F = 128
G = 64
PAD = 10240
RPS = PAD // 16
E_ROWS = 2560
NC, NS = 2, 16

_mesh = plsc.VectorSubcoreMesh(
    core_axis_name="c", subcore_axis_name="s", num_cores=NC, num_subcores=NS)


DEG_RPW = E_ROWS // (NC * NS)


@functools.partial(
    pl.kernel,
    out_type=jax.ShapeDtypeStruct((NC, PAD), jnp.float32),
    mesh=_mesh,
    scratch_types=[
        pltpu.VMEM_SHARED((PAD,), jnp.float32),
        pltpu.VMEM((DEG_RPW, 128), jnp.int32),
        pltpu.VMEM((RPS,), jnp.float32),
        pltpu.VMEM((128,), jnp.float32),
    ],
)
def _sc_degree(dst_h, deg_h, acc_sh, dst_t, zbuf, ones_t):
    c = lax.axis_index("c")
    s = lax.axis_index("s")
    w = c * NS + s

    def fill(i, _):
        zbuf[pl.ds(i * 16, 16)] = jnp.zeros((16,), jnp.float32)
        ones_t[pl.ds((i % 8) * 16, 16)] = jnp.ones((16,), jnp.float32)
        return 0
    lax.fori_loop(0, RPS // 16, fill, 0)
    pltpu.sync_copy(zbuf, acc_sh.at[pl.ds(s * RPS, RPS)])
    pltpu.sync_copy(dst_h.at[pl.ds(w * DEG_RPW, DEG_RPW)], dst_t)

    plsc.subcore_barrier()

    def body(j, _):
        pltpu.sync_copy(ones_t, acc_sh.at[dst_t.at[j]], add=True)
        return 0
    lax.fori_loop(0, DEG_RPW, body, 0)

    plsc.subcore_barrier()
    pltpu.sync_copy(acc_sh.at[pl.ds(s * RPS, RPS)], deg_h.at[c, pl.ds(s * RPS, RPS)])


E_ROWS64 = E_ROWS * 2
AGG_RPW = E_ROWS64 // (NC * NS)
IDXC = 40
NBUF = 4


@functools.partial(
    pl.kernel,
    out_type=jax.ShapeDtypeStruct((NC, PAD, F), jnp.float32),
    mesh=_mesh,
    scratch_types=[
        pltpu.VMEM_SHARED((PAD, F), jnp.float32),
        pltpu.VMEM((IDXC, 64), jnp.int32),
        pltpu.VMEM((IDXC, 64), jnp.int32),
        [pltpu.VMEM((64, F), jnp.float32)] * NBUF,
        [pltpu.SemaphoreType.DMA] * NBUF,
        [pltpu.SemaphoreType.DMA] * NBUF,
    ],
)
def _sc_aggregate(g_h, src_h, dst_h, out_h,
                  acc_sh, src_t, dst_t, rows, gsem, ssem):
    c = lax.axis_index("c")
    s = lax.axis_index("s")

    pltpu.sync_copy(g_h.at[pl.ds(s * RPS, RPS)], acc_sh.at[pl.ds(s * RPS, RPS)])

    plsc.subcore_barrier()

    base = (c * NS + s) * AGG_RPW

    def drain(k):
        pltpu.make_async_copy(rows[k], acc_sh.at[dst_t.at[0]], ssem[k]).wait()

    def outer(blk, _):
        @pl.when(blk > 0)
        def _():
            for k in range(NBUF):
                drain(k)
        off = pl.multiple_of(base + blk * IDXC, 8)
        pltpu.sync_copy(src_h.at[pl.ds(off, IDXC)], src_t)
        pltpu.sync_copy(dst_h.at[pl.ds(off, IDXC)], dst_t)

        def quad(qj, _):
            j0 = qj * NBUF

            gds = []
            for k in range(NBUF):
                @pl.when(qj > 0)
                def _(k=k):
                    drain(k)
                gds.append(
                    pltpu.async_copy(g_h.at[src_t.at[j0 + k]], rows[k], gsem[k]))
            for k in range(NBUF):
                gds[k].wait()
                pltpu.async_copy(rows[k], acc_sh.at[dst_t.at[j0 + k]],
                                 ssem[k], add=True)
            return 0
        lax.fori_loop(0, IDXC // NBUF, quad, 0)
        return 0
    lax.fori_loop(0, AGG_RPW // IDXC, outer, 0)

    for k in range(NBUF):
        drain(k)

    plsc.subcore_barrier()
    pltpu.sync_copy(acc_sh.at[pl.ds(s * RPS, RPS)],
                    out_h.at[c, pl.ds(s * RPS, RPS)])


BLK = PAD // 8


def _prep_body(deg_ref, x_ref, w_ref, g_ref, dinv_ref):
    tot = deg_ref[0] + deg_ref[1] + 1.0
    di = lax.rsqrt(tot)
    h = jnp.dot(x_ref[...], w_ref[...], preferred_element_type=jnp.float32)
    g_ref[...] = h * di[:, None]
    dinv_ref[0, 0] = di


def _tc_prep(deg, x_pad, W1):
    return pl.pallas_call(
        _prep_body,
        grid=(8,),
        in_specs=[
            pl.BlockSpec((NC, BLK), lambda i: (0, i)),
            pl.BlockSpec((BLK, F), lambda i: (i, 0)),
            pl.BlockSpec((F, F), lambda i: (0, 0)),
        ],
        out_specs=[
            pl.BlockSpec((BLK, F), lambda i: (i, 0)),
            pl.BlockSpec((1, 1, BLK), lambda i: (i, 0, 0)),
        ],
        out_shape=[
            jax.ShapeDtypeStruct((PAD, F), jnp.float32),
            jax.ShapeDtypeStruct((8, 1, BLK), jnp.float32),
        ],
    )(deg, x_pad, W1)


def _mid_body(s_ref, g_ref, dinv_ref, b_ref, w_ref, gn_ref):
    tot = s_ref[0] + s_ref[1] - g_ref[...]
    di = dinv_ref[0, 0]
    act = jnp.maximum(tot * di[:, None] + b_ref[0][None, :], 0.0)
    h = jnp.dot(act, w_ref[...], preferred_element_type=jnp.float32)
    gn_ref[...] = h * di[:, None]


def _tc_mid(s_agg, g, dinv, b_prev, W_next):
    return pl.pallas_call(
        _mid_body,
        grid=(8,),
        in_specs=[
            pl.BlockSpec((NC, BLK, F), lambda i: (0, i, 0)),
            pl.BlockSpec((BLK, F), lambda i: (i, 0)),
            pl.BlockSpec((1, 1, BLK), lambda i: (i, 0, 0)),
            pl.BlockSpec((1, F), lambda i: (0, 0)),
            pl.BlockSpec((F, F), lambda i: (0, 0)),
        ],
        out_specs=pl.BlockSpec((BLK, F), lambda i: (i, 0)),
        out_shape=jax.ShapeDtypeStruct((PAD, F), jnp.float32),
    )(s_agg, g, dinv, b_prev.reshape(1, F), W_next)


def _tail_body(s_ref, g_ref, dinv_ref, b_ref, batch_ref, fw1_ref, fb1_ref,
               fw2_ref, fb2_ref, out_ref, sums_sc, cnts_sc):
    i = pl.program_id(0)
    tot = s_ref[0] + s_ref[1] - g_ref[...]
    di = dinv_ref[0, 0]
    act = jnp.maximum(tot * di[:, None] + b_ref[0][None, :], 0.0)
    bids = batch_ref[0, 0]
    onehot = (bids[:, None] ==
              lax.broadcasted_iota(jnp.int32, (BLK, G), 1)).astype(jnp.float32)
    part = lax.dot_general(onehot, act, (((0,), (0,)), ((), ())),
                           preferred_element_type=jnp.float32)
    cnt = jnp.sum(onehot, axis=0)

    @pl.when(i == 0)
    def _():
        sums_sc[...] = jnp.zeros_like(sums_sc)
        cnts_sc[...] = jnp.zeros_like(cnts_sc)
    sums_sc[...] += part
    cnts_sc[0] += cnt

    @pl.when(i == 7)
    def _():
        pooled = sums_sc[...] / jnp.maximum(cnts_sc[0], 1.0)[:, None]
        o1 = jnp.maximum(
            jnp.dot(pooled, fw1_ref[...], preferred_element_type=jnp.float32)
            + fb1_ref[0][None, :], 0.0)
        o2 = (jnp.dot(o1, fw2_ref[...], preferred_element_type=jnp.float32)
              + fb2_ref[0][None, :])
        out_ref[...] = o2


def _tc_tail(s_agg, g, dinv, b3, batch_pad, fcW1, fcb1, fcW2, fcb2):
    return pl.pallas_call(
        _tail_body,
        grid=(8,),
        in_specs=[
            pl.BlockSpec((NC, BLK, F), lambda i: (0, i, 0)),
            pl.BlockSpec((BLK, F), lambda i: (i, 0)),
            pl.BlockSpec((1, 1, BLK), lambda i: (i, 0, 0)),
            pl.BlockSpec((1, F), lambda i: (0, 0)),
            pl.BlockSpec((1, 1, BLK), lambda i: (i, 0, 0)),
            pl.BlockSpec((F, 32), lambda i: (0, 0)),
            pl.BlockSpec((1, 32), lambda i: (0, 0)),
            pl.BlockSpec((32, 1), lambda i: (0, 0)),
            pl.BlockSpec((1, 1), lambda i: (0, 0)),
        ],
        out_specs=pl.BlockSpec((G, 1), lambda i: (0, 0)),
        out_shape=jax.ShapeDtypeStruct((G, 1), jnp.float32),
        scratch_shapes=[
            pltpu.VMEM((G, F), jnp.float32),
            pltpu.VMEM((1, G), jnp.float32),
        ],
    )(s_agg, g, dinv, b3.reshape(1, F), batch_pad, fcW1,
      fcb1.reshape(1, 32), fcW2, fcb2.reshape(1, 1))


def kernel(x, edge_index, edge_attr, batch,
           W1, b1, W2, b2, W3, b3, fcW1, fcb1, fcW2, fcb2):
    n_fill = E_ROWS * 128 - E
    fill = (N + jnp.arange(n_fill, dtype=jnp.int32) % (PAD - N))
    src = jnp.concatenate([edge_index[0], fill]).reshape(E_ROWS64, 64)
    dst = jnp.concatenate([edge_index[1], fill]).reshape(E_ROWS64, 64)

    deg = _sc_degree(dst.reshape(E_ROWS, 128))
    x_pad = jnp.zeros((PAD, F), jnp.float32).at[:N].set(x)
    g1, dinv = _tc_prep(deg, x_pad, W1)

    s1 = _sc_aggregate(g1, src, dst)
    g2 = _tc_mid(s1, g1, dinv, b1, W2)
    s2 = _sc_aggregate(g2, src, dst)
    g3 = _tc_mid(s2, g2, dinv, b2, W3)
    s3 = _sc_aggregate(g3, src, dst)

    batch_pad = jnp.full((PAD,), G, jnp.int32).at[:N].set(batch).reshape(8, 1, BLK)
    return _tc_tail(s3, g3, dinv, b3, batch_pad, fcW1, fcb1, fcW2, fcb2)


# speedup vs baseline: 27.2351x; 1.0061x over previous
import functools

import jax
import jax.numpy as jnp
from jax import lax
from jax.experimental import pallas as pl
from jax.experimental.pallas import tpu as pltpu
from jax.experimental.pallas import tpu_sc as plsc

N = 10000
E = 320000
F = 128
G = 64
PAD = 10240
RPS = PAD // 16
E_ROWS = 2560
NC, NS = 2, 16

_mesh = plsc.VectorSubcoreMesh(
    core_axis_name="c", subcore_axis_name="s", num_cores=NC, num_subcores=NS)


DEG_RPW = E_ROWS // (NC * NS)


@functools.partial(
    pl.kernel,
    out_type=jax.ShapeDtypeStruct((NC, PAD), jnp.float32),
    mesh=_mesh,
    scratch_types=[
        pltpu.VMEM_SHARED((PAD,), jnp.float32),
        pltpu.VMEM((DEG_RPW, 128), jnp.int32),
        pltpu.VMEM((RPS,), jnp.float32),
        pltpu.VMEM((128,), jnp.float32),
    ],
)
def _sc_degree(dst_h, deg_h, acc_sh, dst_t, zbuf, ones_t):
    c = lax.axis_index("c")
    s = lax.axis_index("s")
    w = c * NS + s

    def fill(i, _):
        zbuf[pl.ds(i * 16, 16)] = jnp.zeros((16,), jnp.float32)
        ones_t[pl.ds((i % 8) * 16, 16)] = jnp.ones((16,), jnp.float32)
        return 0
    lax.fori_loop(0, RPS // 16, fill, 0)
    pltpu.sync_copy(zbuf, acc_sh.at[pl.ds(s * RPS, RPS)])
    pltpu.sync_copy(dst_h.at[pl.ds(w * DEG_RPW, DEG_RPW)], dst_t)

    plsc.subcore_barrier()

    def body(j, _):
        pltpu.sync_copy(ones_t, acc_sh.at[dst_t.at[j]], add=True)
        return 0
    lax.fori_loop(0, DEG_RPW, body, 0)

    plsc.subcore_barrier()
    pltpu.sync_copy(acc_sh.at[pl.ds(s * RPS, RPS)], deg_h.at[c, pl.ds(s * RPS, RPS)])


E_ROWS64 = E_ROWS * 2
AGG_RPW = E_ROWS64 // (NC * NS)
IDXC = 16
NCHUNK = AGG_RPW // IDXC
NBUF = 4


@functools.partial(
    pl.kernel,
    out_type=jax.ShapeDtypeStruct((NC, PAD, F), jnp.float32),
    mesh=_mesh,
    scratch_types=[
        pltpu.VMEM_SHARED((PAD, F), jnp.float32),
        [pltpu.VMEM((IDXC, 64), jnp.int32)] * 2,
        [pltpu.VMEM((IDXC, 64), jnp.int32)] * 2,
        [pltpu.VMEM((64, F), jnp.float32)] * NBUF,
        [pltpu.SemaphoreType.DMA] * 2,
        [pltpu.SemaphoreType.DMA] * NBUF,
        [pltpu.SemaphoreType.DMA] * NBUF,
        pltpu.SemaphoreType.DMA,
    ],
)
def _sc_aggregate(g_h, src_h, dst_h, out_h,
                  acc_sh, src_t, dst_t, rows, lsem, gsem, ssem, isem):
    c = lax.axis_index("c")
    s = lax.axis_index("s")
    base = (c * NS + s) * AGG_RPW

    def load_idx(blk, b):
        off = pl.multiple_of(base + blk * IDXC, 8)
        pltpu.async_copy(src_h.at[pl.ds(off, IDXC)], src_t[b], lsem[b])
        pltpu.async_copy(dst_h.at[pl.ds(off, IDXC)], dst_t[b], lsem[b])

    def wait_idx(b):
        pltpu.make_async_copy(src_h.at[pl.ds(0, IDXC)], src_t[b], lsem[b]).wait()
        pltpu.make_async_copy(dst_h.at[pl.ds(0, IDXC)], dst_t[b], lsem[b]).wait()

    def drain(k, b):
        pltpu.make_async_copy(rows[k], acc_sh.at[dst_t[b].at[0]], ssem[k]).wait()

    init = pltpu.async_copy(g_h.at[pl.ds(s * RPS, RPS)],
                            acc_sh.at[pl.ds(s * RPS, RPS)], isem)
    load_idx(0, 0)
    init.wait()
    plsc.subcore_barrier()

    def run_chunk(blk, b):
        wait_idx(b)
        @pl.when(blk > 0)
        def _():
            for k in range(NBUF):
                drain(k, 1 - b)

        @pl.when(blk < NCHUNK - 1)
        def _():
            load_idx(blk + 1, 1 - b)

        def quad(qj, _):
            j0 = qj * NBUF
            gds = []
            for k in range(NBUF):
                @pl.when(qj > 0)
                def _(k=k):
                    drain(k, b)
                gds.append(pltpu.async_copy(
                    g_h.at[src_t[b].at[j0 + k]], rows[k], gsem[k]))
            for k in range(NBUF):
                gds[k].wait()
                pltpu.async_copy(rows[k], acc_sh.at[dst_t[b].at[j0 + k]],
                                 ssem[k], add=True)
            return 0
        lax.fori_loop(0, IDXC // NBUF, quad, 0)

    def pair(bb, _):
        run_chunk(bb * 2, 0)
        run_chunk(bb * 2 + 1, 1)
        return 0
    lax.fori_loop(0, NCHUNK // 2, pair, 0)

    for k in range(NBUF):
        drain(k, 1)

    plsc.subcore_barrier()
    pltpu.sync_copy(acc_sh.at[pl.ds(s * RPS, RPS)],
                    out_h.at[c, pl.ds(s * RPS, RPS)])


BLK = PAD // 8


def _prep_body(deg_ref, x_ref, w_ref, g_ref, dinv_ref):
    tot = deg_ref[0] + deg_ref[1] + 1.0
    di = lax.rsqrt(tot)
    h = jnp.dot(x_ref[...], w_ref[...], preferred_element_type=jnp.float32)
    g_ref[...] = h * di[:, None]
    dinv_ref[0, 0] = di


def _tc_prep(deg, x_pad, W1):
    return pl.pallas_call(
        _prep_body,
        grid=(8,),
        in_specs=[
            pl.BlockSpec((NC, BLK), lambda i: (0, i)),
            pl.BlockSpec((BLK, F), lambda i: (i, 0)),
            pl.BlockSpec((F, F), lambda i: (0, 0)),
        ],
        out_specs=[
            pl.BlockSpec((BLK, F), lambda i: (i, 0)),
            pl.BlockSpec((1, 1, BLK), lambda i: (i, 0, 0)),
        ],
        out_shape=[
            jax.ShapeDtypeStruct((PAD, F), jnp.float32),
            jax.ShapeDtypeStruct((8, 1, BLK), jnp.float32),
        ],
    )(deg, x_pad, W1)


def _mid_body(s_ref, g_ref, dinv_ref, b_ref, w_ref, gn_ref):
    tot = s_ref[0] + s_ref[1] - g_ref[...]
    di = dinv_ref[0, 0]
    act = jnp.maximum(tot * di[:, None] + b_ref[0][None, :], 0.0)
    h = jnp.dot(act, w_ref[...], preferred_element_type=jnp.float32)
    gn_ref[...] = h * di[:, None]


def _tc_mid(s_agg, g, dinv, b_prev, W_next):
    return pl.pallas_call(
        _mid_body,
        grid=(8,),
        in_specs=[
            pl.BlockSpec((NC, BLK, F), lambda i: (0, i, 0)),
            pl.BlockSpec((BLK, F), lambda i: (i, 0)),
            pl.BlockSpec((1, 1, BLK), lambda i: (i, 0, 0)),
            pl.BlockSpec((1, F), lambda i: (0, 0)),
            pl.BlockSpec((F, F), lambda i: (0, 0)),
        ],
        out_specs=pl.BlockSpec((BLK, F), lambda i: (i, 0)),
        out_shape=jax.ShapeDtypeStruct((PAD, F), jnp.float32),
    )(s_agg, g, dinv, b_prev.reshape(1, F), W_next)


def _tail_body(s_ref, g_ref, dinv_ref, b_ref, batch_ref, fw1_ref, fb1_ref,
               fw2_ref, fb2_ref, out_ref, sums_sc, cnts_sc):
    i = pl.program_id(0)
    tot = s_ref[0] + s_ref[1] - g_ref[...]
    di = dinv_ref[0, 0]
    act = jnp.maximum(tot * di[:, None] + b_ref[0][None, :], 0.0)
    bids = batch_ref[0, 0]
    onehot = (bids[:, None] ==
              lax.broadcasted_iota(jnp.int32, (BLK, G), 1)).astype(jnp.float32)
    part = lax.dot_general(onehot, act, (((0,), (0,)), ((), ())),
                           preferred_element_type=jnp.float32)
    cnt = jnp.sum(onehot, axis=0)

    @pl.when(i == 0)
    def _():
        sums_sc[...] = jnp.zeros_like(sums_sc)
        cnts_sc[...] = jnp.zeros_like(cnts_sc)
    sums_sc[...] += part
    cnts_sc[0] += cnt

    @pl.when(i == 7)
    def _():
        pooled = sums_sc[...] / jnp.maximum(cnts_sc[0], 1.0)[:, None]
        o1 = jnp.maximum(
            jnp.dot(pooled, fw1_ref[...], preferred_element_type=jnp.float32)
            + fb1_ref[0][None, :], 0.0)
        o2 = (jnp.dot(o1, fw2_ref[...], preferred_element_type=jnp.float32)
              + fb2_ref[0][None, :])
        out_ref[...] = o2


def _tc_tail(s_agg, g, dinv, b3, batch_pad, fcW1, fcb1, fcW2, fcb2):
    return pl.pallas_call(
        _tail_body,
        grid=(8,),
        in_specs=[
            pl.BlockSpec((NC, BLK, F), lambda i: (0, i, 0)),
            pl.BlockSpec((BLK, F), lambda i: (i, 0)),
            pl.BlockSpec((1, 1, BLK), lambda i: (i, 0, 0)),
            pl.BlockSpec((1, F), lambda i: (0, 0)),
            pl.BlockSpec((1, 1, BLK), lambda i: (i, 0, 0)),
            pl.BlockSpec((F, 32), lambda i: (0, 0)),
            pl.BlockSpec((1, 32), lambda i: (0, 0)),
            pl.BlockSpec((32, 1), lambda i: (0, 0)),
            pl.BlockSpec((1, 1), lambda i: (0, 0)),
        ],
        out_specs=pl.BlockSpec((G, 1), lambda i: (0, 0)),
        out_shape=jax.ShapeDtypeStruct((G, 1), jnp.float32),
        scratch_shapes=[
            pltpu.VMEM((G, F), jnp.float32),
            pltpu.VMEM((1, G), jnp.float32),
        ],
    )(s_agg, g, dinv, b3.reshape(1, F), batch_pad, fcW1,
      fcb1.reshape(1, 32), fcW2, fcb2.reshape(1, 1))


def kernel(x, edge_index, edge_attr, batch,
           W1, b1, W2, b2, W3, b3, fcW1, fcb1, fcW2, fcb2):
    n_fill = E_ROWS * 128 - E
    fill = (N + jnp.arange(n_fill, dtype=jnp.int32) % (PAD - N))
    src = jnp.concatenate([edge_index[0], fill]).reshape(E_ROWS64, 64)
    dst = jnp.concatenate([edge_index[1], fill]).reshape(E_ROWS64, 64)

    deg = _sc_degree(dst.reshape(E_ROWS, 128))
    x_pad = jnp.zeros((PAD, F), jnp.float32).at[:N].set(x)
    g1, dinv = _tc_prep(deg, x_pad, W1)

    s1 = _sc_aggregate(g1, src, dst)
    g2 = _tc_mid(s1, g1, dinv, b1, W2)
    s2 = _sc_aggregate(g2, src, dst)
    g3 = _tc_mid(s2, g2, dinv, b2, W3)
    s3 = _sc_aggregate(g3, src, dst)

    batch_pad = jnp.full((PAD,), G, jnp.int32).at[:N].set(batch).reshape(8, 1, BLK)
    return _tc_tail(s3, g3, dinv, b3, batch_pad, fcW1, fcb1, fcW2, fcb2)
